# baseline (speedup 1.0000x reference)
import functools

import jax
import jax.numpy as jnp
from jax import lax
from jax.experimental import pallas as pl
from jax.experimental.pallas import tpu as pltpu
from jax.experimental.pallas import tpu_sc as plsc

_NT = 50
_LANES = 16


def _stage_a_sc(w, ids, nh, ell):
    b = ids.shape[0]
    nbh = b * nh
    mesh = plsc.VectorSubcoreMesh(core_axis_name="c", subcore_axis_name="s")

    @functools.partial(
        pl.kernel,
        mesh=mesh,
        out_type=jax.ShapeDtypeStruct((nbh * _NT * ell,), jnp.float32),
        compiler_params=pltpu.CompilerParams(needs_layout_passes=False),
        scratch_types=[
            pltpu.VMEM((_NT * _NT,), jnp.float32),
            pltpu.VMEM((ell,), jnp.int32),
            pltpu.VMEM((_NT * ell,), jnp.float32),
        ],
    )
    def k(w_hbm, ids_hbm, t_all_hbm, wh_v, ids_v, row_v):
        cid = lax.axis_index("c")
        sid = lax.axis_index("s")
        wid = sid * 2 + cid

        @pl.when(wid < nbh)
        def _():
            bb = wid // nh
            hh = wid - bb * nh
            pltpu.sync_copy(w_hbm.at[hh], wh_v)
            pltpu.sync_copy(ids_hbm.at[bb], ids_v)

            def t_loop(t, carry):
                def j_loop(jv, c2):
                    tag = ids_v[pl.ds(jv * _LANES, _LANES)]
                    idx = tag + t * _NT
                    val = plsc.load_gather(wh_v, [idx])
                    row_v[pl.ds(t * ell + jv * _LANES, _LANES)] = val
                    return c2

                return lax.fori_loop(0, ell // _LANES, j_loop, carry)

            lax.fori_loop(0, _NT, t_loop, 0)
            pltpu.sync_copy(row_v, t_all_hbm.at[pl.ds(wid * _NT * ell, _NT * ell)])

    return k(w, ids)


def _tc_body(idsi_ref, p_ref, out_ref):
    ti = out_ref.shape[2]
    idsi = idsi_ref[0]
    p = p_ref[0, 0]
    t_iota = jax.lax.broadcasted_iota(jnp.int32, (_NT, ti), 0)
    oit = (idsi == t_iota).astype(jnp.float32)
    out_ref[0, 0] = jax.lax.dot_general(
        oit, p, (((0,), (0,)), ((), ())),
        preferred_element_type=jnp.float32)


def kernel(postag_ids, bias_table):
    ids = postag_ids.astype(jnp.int32)
    b, ell = ids.shape
    nh = bias_table.shape[1]
    w = bias_table.T.reshape(nh, _NT * _NT)

    t_all = _stage_a_sc(w, ids, nh, ell).reshape(b, nh, _NT, ell)

    ti = 1024
    ids3 = ids.reshape(b, 1, ell)
    grid = (b, nh, ell // ti)
    return pl.pallas_call(
        _tc_body,
        grid=grid,
        in_specs=[
            pl.BlockSpec((1, 1, ti), lambda bb, hh, it: (bb, 0, it)),
            pl.BlockSpec((1, 1, _NT, ell), lambda bb, hh, it: (bb, hh, 0, 0)),
        ],
        out_specs=pl.BlockSpec((1, 1, ti, ell),
                               lambda bb, hh, it: (bb, hh, it, 0)),
        out_shape=jax.ShapeDtypeStruct((b, nh, ell, ell), jnp.float32),
    )(ids3, t_all)


# speedup vs baseline: 1.1719x; 1.1719x over previous
import functools

import jax
import jax.numpy as jnp
from jax import lax
from jax.experimental import pallas as pl
from jax.experimental.pallas import tpu as pltpu
from jax.experimental.pallas import tpu_sc as plsc

_NT = 50
_NTP = 128
_LANES = 16
_CHUNK = 128


def _stage_a_sc(wt, ids, nh, ell):
    b = ids.shape[0]
    nbh = b * nh
    pr = 512
    npass = ell // pr
    nchunk = pr // _CHUNK
    mesh = plsc.VectorSubcoreMesh(core_axis_name="c", subcore_axis_name="s")

    @functools.partial(
        pl.kernel,
        mesh=mesh,
        out_type=jax.ShapeDtypeStruct((nbh * ell, _NTP), jnp.float32),
        compiler_params=pltpu.CompilerParams(needs_layout_passes=False),
        scratch_types=[
            pltpu.VMEM((ell,), jnp.int32),
            pltpu.VMEM((nchunk, _CHUNK), jnp.int32),
            pltpu.VMEM((pr, _NTP), jnp.float32),
            pltpu.SemaphoreType.DMA,
        ],
    )
    def k(wt_hbm, ids_hbm, tp_hbm, ids_v, idx_v, rows_v, sem):
        cid = lax.axis_index("c")
        sid = lax.axis_index("s")
        wid = sid * 2 + cid

        @pl.when(wid < nbh)
        def _():
            bb = wid // nh
            hh = wid - bb * nh
            pltpu.sync_copy(ids_hbm.at[bb], ids_v)
            for p in range(npass):
                def c_loop(c, carry):
                    def v_loop(v, c2):
                        tag = ids_v[pl.ds(p * pr + c * _CHUNK + v * _LANES,
                                          _LANES)]
                        idx_v[c, pl.ds(v * _LANES, _LANES)] = tag + hh * _NT
                        return c2
                    return lax.fori_loop(0, _CHUNK // _LANES, v_loop, carry)
                lax.fori_loop(0, nchunk, c_loop, 0)
                copies = [
                    pltpu.async_copy(
                        wt_hbm.at[idx_v.at[c]],
                        rows_v.at[pl.ds(c * _CHUNK, _CHUNK)],
                        sem,
                    )
                    for c in range(nchunk)
                ]
                for cp in copies:
                    cp.wait()
                pltpu.sync_copy(
                    rows_v,
                    tp_hbm.at[pl.ds(wid * ell + p * pr, pr)],
                )

    return k(wt, ids)


def _tc_body(idsi_ref, p_ref, out_ref):
    ti = out_ref.shape[2]
    idsi = idsi_ref[0]
    p2 = p_ref[0, 0]
    t_iota = jax.lax.broadcasted_iota(jnp.int32, (_NTP, ti), 0)
    oit = (idsi == t_iota).astype(jnp.float32)
    out_ref[0, 0] = jax.lax.dot_general(
        oit, p2, (((0,), (1,)), ((), ())),
        preferred_element_type=jnp.float32)


def kernel(postag_ids, bias_table):
    ids = postag_ids.astype(jnp.int32)
    b, ell = ids.shape
    nh = bias_table.shape[1]
    wt = jnp.transpose(bias_table.reshape(_NT, _NT, nh), (2, 1, 0))
    wt = jnp.pad(wt, ((0, 0), (0, 0), (0, _NTP - _NT))).reshape(nh * _NT, _NTP)

    tp = _stage_a_sc(wt, ids, nh, ell).reshape(b, nh, ell, _NTP)

    ti = 1024
    ids3 = ids.reshape(b, 1, ell)
    grid = (b, nh, ell // ti)
    return pl.pallas_call(
        _tc_body,
        grid=grid,
        in_specs=[
            pl.BlockSpec((1, 1, ti), lambda bb, hh, it: (bb, 0, it)),
            pl.BlockSpec((1, 1, ell, _NTP), lambda bb, hh, it: (bb, hh, 0, 0)),
        ],
        out_specs=pl.BlockSpec((1, 1, ti, ell),
                               lambda bb, hh, it: (bb, hh, it, 0)),
        out_shape=jax.ShapeDtypeStruct((b, nh, ell, ell), jnp.float32),
    )(ids3, tp)
